# baseline (speedup 1.0000x reference)
import functools

import jax
import jax.numpy as jnp
from jax import lax
from jax.experimental import pallas as pl
from jax.experimental.pallas import tpu as pltpu
from jax.experimental.pallas import tpu_sc as plsc

B = 128
I = 512
O = 512
W = 512
K = 154
WB = 8
GRID = W // WB

NC = 2
NS = 16
NW = NC * NS
TPW = B // NW
NV = W // 16


def _gate_body(x_ref, gw_ref, gb_ref, logits_ref, keys_ref):
    logits = jax.lax.dot_general(
        x_ref[...], gw_ref[...], (((1,), (1,)), ((), ())),
        preferred_element_type=jnp.float32,
    ) + gb_ref[...]
    logits_ref[...] = logits
    bits = jax.lax.bitcast_convert_type(logits, jnp.int32)
    keys_ref[...] = jnp.where(
        bits >= 0, bits, (~bits) ^ jnp.int32(-2147483648)
    )


def _route_sc_body(logits_hbm, keys_hbm, out_hbm, row_v, key_v, out_v):
    wid = lax.axis_index("s") * NC + lax.axis_index("c")
    base = wid * TPW
    pltpu.sync_copy(logits_hbm.at[pl.ds(base, TPW)], row_v)
    pltpu.sync_copy(keys_hbm.at[pl.ds(base, TPW)], key_v)

    def _splat(v, op):
        for sh in (8, 4, 2, 1):
            perm = lax.iota(jnp.int32, 16) ^ jnp.full(
                (16,), sh, dtype=jnp.int32
            )
            v = op(
                v,
                lax.gather(
                    v,
                    perm[:, None],
                    lax.GatherDimensionNumbers(
                        offset_dims=(), collapsed_slice_dims=(0,),
                        start_index_map=(0,),
                    ),
                    (1,),
                    mode=lax.GatherScatterMode.PROMISE_IN_BOUNDS,
                ),
            )
        return v

    one_v = jnp.ones((16,), jnp.int32)
    zero_v = jnp.zeros((16,), jnp.int32)
    k_v = jnp.full((16,), K, dtype=jnp.int32)

    for t in range(TPW):
        lo = jnp.full((16,), -2147483648, dtype=jnp.int32)
        hi = jnp.full((16,), 2147483647, dtype=jnp.int32)

        def bis(_, carry):
            lo, hi = carry
            mid = (lo >> one_v) + (hi >> one_v) + (lo & hi & one_v)

            def cnt_body(j, cnt):
                kv = key_v[t, pl.ds(j * 16, 16)]
                return cnt + jnp.where(kv >= mid, one_v, zero_v)

            cnt = lax.fori_loop(0, NV, cnt_body, zero_v)
            ge = _splat(cnt, jnp.add) >= k_v
            return jnp.where(ge, mid, lo), jnp.where(ge, hi, mid)

        lo, hi = lax.fori_loop(0, 32, bis, (lo, hi))

        def mx_body(j, mx):
            return jnp.maximum(mx, row_v[t, pl.ds(j * 16, 16)])

        mx = lax.fori_loop(
            0, NV, mx_body, jnp.full((16,), -3.0e38, dtype=jnp.float32)
        )
        mv = _splat(mx, jnp.maximum)

        def e_body(j, den):
            v = row_v[t, pl.ds(j * 16, 16)]
            kv = key_v[t, pl.ds(j * 16, 16)]
            e = jnp.where(
                kv >= lo, jnp.exp(v - mv), jnp.zeros((16,), jnp.float32)
            )
            out_v[t, pl.ds(j * 16, 16)] = e
            return den + e

        den = lax.fori_loop(0, NV, e_body, jnp.zeros((16,), jnp.float32))
        inv = jnp.ones((16,), jnp.float32) / _splat(den, jnp.add)

        def sc_body(j, _):
            out_v[t, pl.ds(j * 16, 16)] = out_v[t, pl.ds(j * 16, 16)] * inv
            return 0

        lax.fori_loop(0, NV, sc_body, 0)

    pltpu.sync_copy(out_v, out_hbm.at[pl.ds(base, TPW)])


def _moe_body(x_ref, dw_ref, tiles_ref, out_ref, dwt_scr, acc_scr):
    i = pl.program_id(0)

    @pl.when(i == 0)
    def _prologue():
        dwt_scr[...] = dw_ref[...].T
        acc_scr[...] = jnp.zeros_like(acc_scr)

    @pl.when(i > 0)
    def _accum():
        x = x_ref[...]
        blk = i - 1
        dwb = dwt_scr[pl.ds(blk * WB, WB), :]
        acc = acc_scr[...]
        for j in range(WB):
            t = tiles_ref[j]
            pt = jax.lax.dot_general(
                t, x, (((1,), (1,)), ((), ())),
                preferred_element_type=jnp.float32,
            )
            acc = acc + dwb[j : j + 1, :] * pt
        acc_scr[...] = acc

    @pl.when(i == GRID)
    def _final():
        out_ref[...] = acc_scr[...].T


def kernel(x, gate_w, gate_b, tiles):
    logits, keys = pl.pallas_call(
        _gate_body,
        in_specs=[
            pl.BlockSpec((B, I), lambda: (0, 0)),
            pl.BlockSpec((W, I), lambda: (0, 0)),
            pl.BlockSpec((1, W), lambda: (0, 0)),
        ],
        out_specs=[
            pl.BlockSpec((B, W), lambda: (0, 0)),
            pl.BlockSpec((B, W), lambda: (0, 0)),
        ],
        out_shape=[
            jax.ShapeDtypeStruct((B, W), jnp.float32),
            jax.ShapeDtypeStruct((B, W), jnp.int32),
        ],
    )(x, gate_w, gate_b.reshape(1, W))

    mesh = plsc.VectorSubcoreMesh(core_axis_name="c", subcore_axis_name="s")
    dw = pl.kernel(
        _route_sc_body,
        mesh=mesh,
        out_type=jax.ShapeDtypeStruct((B, W), jnp.float32),
        scratch_types=[
            pltpu.VMEM((TPW, W), jnp.float32),
            pltpu.VMEM((TPW, W), jnp.int32),
            pltpu.VMEM((TPW, W), jnp.float32),
        ],
    )(logits, keys)

    out = pl.pallas_call(
        _moe_body,
        grid=(GRID + 1,),
        in_specs=[
            pl.BlockSpec((B, I), lambda i: (0, 0)),
            pl.BlockSpec((B, W), lambda i: (0, 0)),
            pl.BlockSpec(
                (WB, O, I), lambda i: (jnp.maximum(i - 1, 0), 0, 0)
            ),
        ],
        out_specs=pl.BlockSpec((B, O), lambda i: (0, 0)),
        out_shape=jax.ShapeDtypeStruct((B, O), jnp.float32),
        scratch_shapes=[
            pltpu.VMEM((W, B), jnp.float32),
            pltpu.VMEM((O, B), jnp.float32),
        ],
    )(x, dw, tiles)
    return out


# speedup vs baseline: 1.0605x; 1.0605x over previous
import functools

import jax
import jax.numpy as jnp
from jax import lax
from jax.experimental import pallas as pl
from jax.experimental.pallas import tpu as pltpu
from jax.experimental.pallas import tpu_sc as plsc

B = 128
I = 512
O = 512
W = 512
K = 154
WB = 8
GRID = W // WB

NC = 2
NS = 16
NW = NC * NS
TPW = B // NW
NV = W // 16


def _gate_body(x_ref, gw_ref, gb_ref, logits_ref, keys_ref):
    logits = jax.lax.dot_general(
        x_ref[...], gw_ref[...], (((1,), (1,)), ((), ())),
        preferred_element_type=jnp.float32,
    ) + gb_ref[...]
    logits_ref[...] = logits
    bits = jax.lax.bitcast_convert_type(logits, jnp.int32)
    keys_ref[...] = jnp.where(
        bits >= 0, bits, (~bits) ^ jnp.int32(-2147483648)
    )


def _route_sc_body(logits_hbm, keys_hbm, out_hbm, row_v, key_v, out_v):
    wid = lax.axis_index("s") * NC + lax.axis_index("c")
    base = wid * TPW
    pltpu.sync_copy(logits_hbm.at[pl.ds(base, TPW)], row_v)
    pltpu.sync_copy(keys_hbm.at[pl.ds(base, TPW)], key_v)

    def _splat(v, op):
        for sh in (8, 4, 2, 1):
            perm = lax.iota(jnp.int32, 16) ^ jnp.full(
                (16,), sh, dtype=jnp.int32
            )
            v = op(
                v,
                lax.gather(
                    v,
                    perm[:, None],
                    lax.GatherDimensionNumbers(
                        offset_dims=(), collapsed_slice_dims=(0,),
                        start_index_map=(0,),
                    ),
                    (1,),
                    mode=lax.GatherScatterMode.PROMISE_IN_BOUNDS,
                ),
            )
        return v

    one_v = jnp.ones((16,), jnp.int32)
    zero_v = jnp.zeros((16,), jnp.int32)
    k_v = jnp.full((16,), K, dtype=jnp.int32)
    zf_v = jnp.zeros((16,), jnp.float32)
    TS = tuple(range(TPW))

    los = tuple(jnp.full((16,), -2147483648, dtype=jnp.int32) for _ in TS)
    his = tuple(jnp.full((16,), 2147483647, dtype=jnp.int32) for _ in TS)

    def bis(_, carry):
        los, his = carry
        mids = tuple(
            (los[t] >> one_v) + (his[t] >> one_v) + (los[t] & his[t] & one_v)
            for t in TS
        )

        def cnt_body(j, cnts):
            return tuple(
                cnts[t]
                + jnp.where(
                    key_v[t, pl.ds(j * 16, 16)] >= mids[t], one_v, zero_v
                )
                for t in TS
            )

        cnts = lax.fori_loop(
            0, NV, cnt_body, tuple(zero_v for _ in TS), unroll=4
        )
        ges = tuple(_splat(cnts[t], jnp.add) >= k_v for t in TS)
        return (
            tuple(jnp.where(ges[t], mids[t], los[t]) for t in TS),
            tuple(jnp.where(ges[t], his[t], mids[t]) for t in TS),
        )

    los, his = lax.fori_loop(0, 32, bis, (los, his))

    def mx_body(j, mxs):
        return tuple(
            jnp.maximum(mxs[t], row_v[t, pl.ds(j * 16, 16)]) for t in TS
        )

    mxs = lax.fori_loop(
        0, NV, mx_body,
        tuple(jnp.full((16,), -3.0e38, dtype=jnp.float32) for _ in TS),
        unroll=4,
    )
    mvs = tuple(_splat(mxs[t], jnp.maximum) for t in TS)

    def e_body(j, dens):
        outs = []
        for t in TS:
            v = row_v[t, pl.ds(j * 16, 16)]
            kv = key_v[t, pl.ds(j * 16, 16)]
            e = jnp.where(kv >= los[t], jnp.exp(v - mvs[t]), zf_v)
            out_v[t, pl.ds(j * 16, 16)] = e
            outs.append(dens[t] + e)
        return tuple(outs)

    dens = lax.fori_loop(
        0, NV, e_body, tuple(zf_v for _ in TS), unroll=4
    )
    invs = tuple(
        jnp.ones((16,), jnp.float32) / _splat(dens[t], jnp.add) for t in TS
    )

    def sc_body(j, _):
        for t in TS:
            out_v[t, pl.ds(j * 16, 16)] = (
                out_v[t, pl.ds(j * 16, 16)] * invs[t]
            )
        return 0

    lax.fori_loop(0, NV, sc_body, 0, unroll=4)

    pltpu.sync_copy(out_v, out_hbm.at[pl.ds(base, TPW)])


def _moe_body(x_ref, dw_ref, tiles_ref, out_ref, dwt_scr, acc_scr):
    i = pl.program_id(0)

    @pl.when(i == 0)
    def _prologue():
        dwt_scr[...] = dw_ref[...].T
        acc_scr[...] = jnp.zeros_like(acc_scr)

    @pl.when(i > 0)
    def _accum():
        x = x_ref[...]
        blk = i - 1
        dwb = dwt_scr[pl.ds(blk * WB, WB), :]
        acc = acc_scr[...]
        for j in range(WB):
            t = tiles_ref[j]
            pt = jax.lax.dot_general(
                t, x, (((1,), (1,)), ((), ())),
                preferred_element_type=jnp.float32,
            )
            acc = acc + dwb[j : j + 1, :] * pt
        acc_scr[...] = acc

    @pl.when(i == GRID)
    def _final():
        out_ref[...] = acc_scr[...].T


def kernel(x, gate_w, gate_b, tiles):
    logits, keys = pl.pallas_call(
        _gate_body,
        in_specs=[
            pl.BlockSpec((B, I), lambda: (0, 0)),
            pl.BlockSpec((W, I), lambda: (0, 0)),
            pl.BlockSpec((1, W), lambda: (0, 0)),
        ],
        out_specs=[
            pl.BlockSpec((B, W), lambda: (0, 0)),
            pl.BlockSpec((B, W), lambda: (0, 0)),
        ],
        out_shape=[
            jax.ShapeDtypeStruct((B, W), jnp.float32),
            jax.ShapeDtypeStruct((B, W), jnp.int32),
        ],
    )(x, gate_w, gate_b.reshape(1, W))

    mesh = plsc.VectorSubcoreMesh(core_axis_name="c", subcore_axis_name="s")
    dw = pl.kernel(
        _route_sc_body,
        mesh=mesh,
        out_type=jax.ShapeDtypeStruct((B, W), jnp.float32),
        scratch_types=[
            pltpu.VMEM((TPW, W), jnp.float32),
            pltpu.VMEM((TPW, W), jnp.int32),
            pltpu.VMEM((TPW, W), jnp.float32),
        ],
    )(logits, keys)

    out = pl.pallas_call(
        _moe_body,
        grid=(GRID + 1,),
        in_specs=[
            pl.BlockSpec((B, I), lambda i: (0, 0)),
            pl.BlockSpec((B, W), lambda i: (0, 0)),
            pl.BlockSpec(
                (WB, O, I), lambda i: (jnp.maximum(i - 1, 0), 0, 0)
            ),
        ],
        out_specs=pl.BlockSpec((B, O), lambda i: (0, 0)),
        out_shape=jax.ShapeDtypeStruct((B, O), jnp.float32),
        scratch_shapes=[
            pltpu.VMEM((W, B), jnp.float32),
            pltpu.VMEM((O, B), jnp.float32),
        ],
    )(x, dw, tiles)
    return out
